# Initial kernel scaffold; baseline (speedup 1.0000x reference)
import jax
import jax.numpy as jnp
from jax.experimental import pallas as pl


def kernel(x1, edge_index1, ent1, batch1, x2, edge_index2, ent2, batch2, atom_emb, gcn_W1, gcn_b1, gcn_W2, gcn_b2, fc_W, fc_b, ent_emb, enc_W1, enc_b1, enc_W2, enc_b2, dec_W1, dec_b1, dec_W2, dec_b2, dec_W3, dec_b3):
    raise NotImplementedError("write your pallas kernel here")


# speedup vs baseline: 10.6125x; 10.6125x over previous
import functools

import jax
import jax.numpy as jnp
from jax import lax
from jax.experimental import pallas as pl
from jax.experimental.pallas import tpu as pltpu
from jax.experimental.pallas import tpu_sc as plsc

N = 10000
E = 320000
G = 1024
NPAD = 10240
GPAD = 1032
D = 128
NC, NS, L = 2, 16, 16
EPT = E // NS
CH = 80
NCH = EPT // CH
NPT = 624
ZB = 208
BPT = NPAD // NS
BCH = BPT // CH
GPT = G // NS

_f32 = jnp.float32
_i32 = jnp.int32


def _fill_zero(ref, nrows, width):
    @pl.loop(0, nrows)
    def _(i):
        for j in range(width // L):
            ref[i, pl.ds(j * L, L)] = jnp.zeros((L,), _f32)


def _stats_body(dstF, batF, entF, emb, degO, bcntO, entO,
                dacc, bacc, zb1, vbuf, eidx, erows, didx, sem):
    c = lax.axis_index("c")
    s = lax.axis_index("s")
    ebase = c * G + s * GPT
    pltpu.sync_copy(entF.at[pl.ds(ebase, GPT)], eidx)
    pltpu.async_copy(emb.at[eidx], erows, sem).wait()
    pltpu.sync_copy(erows, entO.at[pl.ds(ebase, GPT)])
    @pl.loop(0, NPT // L + 1)
    def _(i):
        zb1[pl.ds(i * L, L)] = jnp.zeros((L,), _f32)
    for i in range(CH // L):
        vbuf[pl.ds(i * L, L)] = jnp.full((L,), 1.0, _f32)
    pltpu.sync_copy(zb1.at[pl.ds(0, NPT)], dacc.at[pl.ds(s * NPT, NPT)])
    pltpu.sync_copy(zb1.at[pl.ds(0, 64)], bacc.at[pl.ds(s * 64, 64)])
    @pl.when(s == NS - 1)
    def _():
        pltpu.sync_copy(zb1.at[pl.ds(0, 16)], dacc.at[pl.ds(NS * NPT, 16)])
    @pl.when(s == 0)
    def _():
        pltpu.sync_copy(zb1.at[pl.ds(0, 8)], bacc.at[pl.ds(G, 8)])
    plsc.subcore_barrier()
    @pl.loop(0, NCH)
    def _(t):
        base = c * E + s * EPT + t * CH
        pltpu.sync_copy(dstF.at[pl.ds(base, CH)], didx)
        pltpu.sync_copy(vbuf, dacc.at[didx], add=True)
    @pl.loop(0, BCH)
    def _(t):
        base = c * NPAD + s * BPT + t * CH
        pltpu.sync_copy(batF.at[pl.ds(base, CH)], didx)
        pltpu.sync_copy(vbuf, bacc.at[didx], add=True)
    plsc.subcore_barrier()
    pltpu.sync_copy(dacc.at[pl.ds(s * NPT, NPT)], zb1.at[pl.ds(0, NPT)])
    pltpu.sync_copy(zb1.at[pl.ds(0, NPT)], degO.at[pl.ds(c * N + s * NPT, NPT)])
    @pl.when(s == NS - 1)
    def _():
        pltpu.sync_copy(dacc.at[pl.ds(NS * NPT, 16)], zb1.at[pl.ds(NPT, 16)])
        pltpu.sync_copy(zb1.at[pl.ds(NPT, 16)],
                        degO.at[pl.ds(c * N + NS * NPT, 16)])
    pltpu.sync_copy(bacc.at[pl.ds(s * 64, 64)], vbuf.at[pl.ds(0, 64)])
    pltpu.sync_copy(vbuf.at[pl.ds(0, 64)],
                    bcntO.at[pl.ds(c * GPAD + s * 64, 64)])
    @pl.when(s == 0)
    def _():
        pltpu.sync_copy(bacc.at[pl.ds(G, 8)], vbuf.at[pl.ds(64, 8)])
        pltpu.sync_copy(vbuf.at[pl.ds(64, 8)],
                        bcntO.at[pl.ds(c * GPAD + G, 8)])


@functools.cache
def _stats_call():
    mesh = plsc.VectorSubcoreMesh(core_axis_name="c", subcore_axis_name="s",
                                  num_cores=NC, num_subcores=NS)
    return pl.kernel(
        _stats_body,
        out_type=[jax.ShapeDtypeStruct((NC * N,), _f32),
                  jax.ShapeDtypeStruct((NC * GPAD,), _f32),
                  jax.ShapeDtypeStruct((NC * G, D), _f32)],
        mesh=mesh,
        scratch_types=[pltpu.VMEM_SHARED((N,), _f32),
                       pltpu.VMEM_SHARED((GPAD,), _f32),
                       pltpu.VMEM((NPT + L,), _f32),
                       pltpu.VMEM((CH,), _f32),
                       pltpu.VMEM((GPT,), _i32),
                       pltpu.VMEM((GPT, D), _f32),
                       pltpu.VMEM((CH,), _i32),
                       pltpu.SemaphoreType.DMA],
    )


def _msg_body(htF, srcF, dstF, SO, sacc, zb, sidx, didx, rows, sem):
    c = lax.axis_index("c")
    s = lax.axis_index("s")
    _fill_zero(zb, ZB, D)
    for k in range(3):
        pltpu.sync_copy(zb.at[pl.ds(0, ZB)],
                        sacc.at[pl.ds(s * NPT + k * ZB, ZB)])
    @pl.when(s == NS - 1)
    def _():
        pltpu.sync_copy(zb.at[pl.ds(0, 16)], sacc.at[pl.ds(NS * NPT, 16)])
    plsc.subcore_barrier()
    @pl.loop(0, NCH)
    def _(t):
        base = c * E + s * EPT + t * CH
        pltpu.sync_copy(srcF.at[pl.ds(base, CH)], sidx)
        pltpu.sync_copy(dstF.at[pl.ds(base, CH)], didx)
        pltpu.async_copy(htF.at[sidx], rows, sem).wait()
        pltpu.sync_copy(rows, sacc.at[didx], add=True)
    plsc.subcore_barrier()
    pltpu.sync_copy(sacc.at[pl.ds(s * NPT, NPT)],
                    SO.at[pl.ds(c * N + s * NPT, NPT)])
    @pl.when(s == NS - 1)
    def _():
        pltpu.sync_copy(sacc.at[pl.ds(NS * NPT, 16)],
                        SO.at[pl.ds(c * N + NS * NPT, 16)])


@functools.cache
def _msg_call():
    mesh = plsc.VectorSubcoreMesh(core_axis_name="c", subcore_axis_name="s",
                                  num_cores=NC, num_subcores=NS)
    return pl.kernel(
        _msg_body,
        out_type=jax.ShapeDtypeStruct((NC * N, D), _f32),
        mesh=mesh,
        scratch_types=[pltpu.VMEM_SHARED((N, D), _f32),
                       pltpu.VMEM((ZB, D), _f32),
                       pltpu.VMEM((CH,), _i32),
                       pltpu.VMEM((CH,), _i32),
                       pltpu.VMEM((CH, D), _f32),
                       pltpu.SemaphoreType.DMA],
    )


def _pool_body(h2F, batF, poolO, pacc, zb, didx, rows):
    c = lax.axis_index("c")
    s = lax.axis_index("s")
    _fill_zero(zb, 64, D)
    pltpu.sync_copy(zb.at[pl.ds(0, 64)], pacc.at[pl.ds(s * 64, 64)])
    @pl.when(s == 0)
    def _():
        pltpu.sync_copy(zb.at[pl.ds(0, 8)], pacc.at[pl.ds(G, 8)])
    plsc.subcore_barrier()
    @pl.loop(0, BCH)
    def _(t):
        base = c * NPAD + s * BPT + t * CH
        pltpu.sync_copy(batF.at[pl.ds(base, CH)], didx)
        pltpu.sync_copy(h2F.at[pl.ds(base, CH)], rows)
        pltpu.sync_copy(rows, pacc.at[didx], add=True)
    plsc.subcore_barrier()
    pltpu.sync_copy(pacc.at[pl.ds(s * 64, 64)],
                    poolO.at[pl.ds(c * GPAD + s * 64, 64)])
    @pl.when(s == 0)
    def _():
        pltpu.sync_copy(pacc.at[pl.ds(G, 8)],
                        poolO.at[pl.ds(c * GPAD + G, 8)])


@functools.cache
def _pool_call():
    mesh = plsc.VectorSubcoreMesh(core_axis_name="c", subcore_axis_name="s",
                                  num_cores=NC, num_subcores=NS)
    return pl.kernel(
        _pool_body,
        out_type=jax.ShapeDtypeStruct((NC * GPAD, D), _f32),
        mesh=mesh,
        scratch_types=[pltpu.VMEM_SHARED((GPAD, D), _f32),
                       pltpu.VMEM((64, D), _f32),
                       pltpu.VMEM((CH,), _i32),
                       pltpu.VMEM((CH, D), _f32)],
    )


RB = 2000


def _prep_tc(degc, xc, atomP, W1, ht_o, dis_o):
    T = jnp.dot(atomP[...], W1[...], preferred_element_type=_f32)
    dis = lax.rsqrt(degc[...] + 1.0)
    dis_o[...] = dis
    oh = (lax.broadcasted_iota(_i32, (RB, 16), 1) == xc[...]).astype(_f32)
    ht_o[...] = dis * jnp.dot(oh, T, preferred_element_type=_f32)


def _mid_tc(S, ht, dis, b1, W2, out):
    h1 = jnp.maximum(dis[...] * (S[...] + ht[...]) + b1[...], 0.0)
    out[...] = dis[...] * jnp.dot(h1, W2[...], preferred_element_type=_f32)


def _act_tc(S, ht, dis, b2, out):
    out[...] = jnp.maximum(dis[...] * (S[...] + ht[...]) + b2[...], 0.0)


def _final_tc(P1, P2, bc1, bc2, er1, er2, eW1, eb1, eW2, eb2,
              fcW, fcb, W1g, W1e, db1, dW2, db2, dW3, db3, out):
    relu = lambda v: jnp.maximum(v, 0.0)
    dot = functools.partial(jnp.dot, preferred_element_type=_f32)
    g1 = dot(P1[...] / jnp.maximum(bc1[...], 1.0), fcW[...]) + fcb[...]
    g2 = dot(P2[...] / jnp.maximum(bc2[...], 1.0), fcW[...]) + fcb[...]
    gl = relu(g1 + g2)
    e1 = relu(er1[...])
    e1 = relu(dot(e1, eW1[...]) + eb1[...])
    e1 = relu(dot(e1, eW2[...]) + eb2[...])
    e2 = relu(er2[...])
    e2 = relu(dot(e2, eW1[...]) + eb1[...])
    e2 = relu(dot(e2, eW2[...]) + eb2[...])
    el = relu(e1 + e2)
    h = relu(dot(gl, W1g[...]) + dot(el, W1e[...]) + db1[...])
    h = relu(dot(h, dW2[...]) + db2[...])
    out[...] = dot(h, dW3[...]) + db3[...]


def _row_spec(w):
    return pl.BlockSpec((RB, w), lambda i: (i, 0))


def _full_spec(r, w):
    return pl.BlockSpec((r, w), lambda i: (0, 0))


_prep_call = pl.pallas_call(
    _prep_tc,
    grid=(NC * N // RB,),
    in_specs=[_row_spec(1), _row_spec(1), _full_spec(16, D), _full_spec(D, D)],
    out_specs=[_row_spec(D), _row_spec(1)],
    out_shape=[jax.ShapeDtypeStruct((NC * N, D), _f32),
               jax.ShapeDtypeStruct((NC * N, 1), _f32)],
)

_mid_call = pl.pallas_call(
    _mid_tc,
    grid=(NC * N // RB,),
    in_specs=[_row_spec(D), _row_spec(D), _row_spec(1),
              _full_spec(1, D), _full_spec(D, D)],
    out_specs=_row_spec(D),
    out_shape=jax.ShapeDtypeStruct((NC * N, D), _f32),
)

_act_call = pl.pallas_call(
    _act_tc,
    grid=(NC * N // RB,),
    in_specs=[_row_spec(D), _row_spec(D), _row_spec(1), _full_spec(1, D)],
    out_specs=_row_spec(D),
    out_shape=jax.ShapeDtypeStruct((NC * N, D), _f32),
)

_final_call = pl.pallas_call(
    _final_tc,
    out_shape=jax.ShapeDtypeStruct((G, D), _f32),
)


def kernel(x1, edge_index1, ent1, batch1, x2, edge_index2, ent2, batch2,
           atom_emb, gcn_W1, gcn_b1, gcn_W2, gcn_b2, fc_W, fc_b,
           ent_emb, enc_W1, enc_b1, enc_W2, enc_b2,
           dec_W1, dec_b1, dec_W2, dec_b2, dec_W3, dec_b3):
    srcF = jnp.concatenate([edge_index1[0], edge_index2[0] + N]).astype(_i32)
    dstF = jnp.concatenate([edge_index1[1], edge_index2[1]]).astype(_i32)
    pad = jnp.full((NPAD - N,), G, _i32)
    batF = jnp.concatenate([batch1.astype(_i32), pad,
                            batch2.astype(_i32), pad])
    entF = jnp.concatenate([ent1, ent2]).astype(_i32)

    degF, bcntF, entO = _stats_call()(dstF, batF, entF, ent_emb)

    degc = degF.reshape(NC * N, 1)
    xc = jnp.concatenate([x1, x2]).astype(_i32).reshape(NC * N, 1)
    atomP = jnp.pad(atom_emb, ((0, 5), (0, 0)))
    htF, disC = _prep_call(degc, xc, atomP, gcn_W1)

    S1 = _msg_call()(htF, srcF, dstF)
    ht2F = _mid_call(S1, htF, disC, gcn_b1.reshape(1, D), gcn_W2)
    S2 = _msg_call()(ht2F, srcF, dstF)
    h2F = _act_call(S2, ht2F, disC, gcn_b2.reshape(1, D))

    zpad = jnp.zeros((NPAD - N, D), _f32)
    h2P = jnp.concatenate([h2F[:N], zpad, h2F[N:], zpad])
    poolF = _pool_call()(h2P, batF)

    return _final_call(
        poolF[:G], poolF[GPAD:GPAD + G],
        bcntF[:G].reshape(G, 1), bcntF[GPAD:GPAD + G].reshape(G, 1),
        entO[:G], entO[G:],
        enc_W1, enc_b1.reshape(1, D), enc_W2, enc_b2.reshape(1, D),
        fc_W, fc_b.reshape(1, D),
        dec_W1[:D], dec_W1[D:], dec_b1.reshape(1, 2 * D),
        dec_W2, dec_b2.reshape(1, 2 * D), dec_W3, dec_b3.reshape(1, D))
